# Initial kernel scaffold; baseline (speedup 1.0000x reference)
import jax
import jax.numpy as jnp
from jax.experimental import pallas as pl


def kernel(g_idx, c_idx, graph_emb, ctx_emb):
    raise NotImplementedError("write your pallas kernel here")


# speedup vs baseline: 1.0961x; 1.0961x over previous
import functools

import jax
import jax.numpy as jnp
from jax import lax
from jax.experimental import pallas as pl
from jax.experimental.pallas import tpu as pltpu
from jax.experimental.pallas import tpu_sc as plsc

NUM_GRAPHS = 100000
CTX_VOCAB = 100000
EMB_DIM = 128
BATCH = 16384

NUM_WORKERS = 32
BPW = BATCH // NUM_WORKERS
CB = 256
LANES = 16


def _sc_body(g_idx_hbm, c_idx_hbm, g_emb_hbm, c_emb_hbm, out_hbm,
             gidx_v, cidx_v, grows_v, crows_v, out_v, sem):
    cid = lax.axis_index("c")
    sid = lax.axis_index("s")
    wid = sid * 2 + cid
    base = wid * BPW

    iota16 = lax.iota(jnp.int32, LANES)

    for ci in range(BPW // CB):
        pltpu.sync_copy(g_idx_hbm.at[pl.ds(base + ci * CB, CB)], gidx_v)
        pltpu.sync_copy(c_idx_hbm.at[pl.ds(base + ci * CB, CB)], cidx_v)
        gcopy = pltpu.async_copy(g_emb_hbm.at[gidx_v], grows_v, sem)
        ccopy = pltpu.async_copy(c_emb_hbm.at[cidx_v], crows_v, sem)
        gcopy.wait()
        ccopy.wait()

        def group_body(g, _, ci=ci):
            res = jnp.zeros((LANES,), jnp.float32)
            for r in range(LANES):
                row = g * LANES + r
                acc = jnp.zeros((LANES,), jnp.float32)
                for j in range(EMB_DIM // LANES):
                    gv = grows_v[row, pl.ds(j * LANES, LANES)]
                    cv = crows_v[row, pl.ds(j * LANES, LANES)]
                    acc = acc + gv * cv
                s = jnp.sum(acc)
                res = jnp.where(iota16 == r, s, res)
            out_v[pl.ds(ci * CB + g * LANES, LANES)] = res
            return 0

        lax.fori_loop(0, CB // LANES, group_body, 0)

    pltpu.sync_copy(out_v, out_hbm.at[pl.ds(base, BPW)])


@functools.partial(jax.jit, static_argnames=())
def _pvdbow_scores(g_idx, c_idx, graph_emb, ctx_emb):
    mesh = plsc.VectorSubcoreMesh(core_axis_name="c", subcore_axis_name="s")
    f = pl.kernel(
        _sc_body,
        out_type=jax.ShapeDtypeStruct((BATCH,), jnp.float32),
        mesh=mesh,
        compiler_params=pltpu.CompilerParams(needs_layout_passes=False),
        scratch_types=[
            pltpu.VMEM((CB,), jnp.int32),
            pltpu.VMEM((CB,), jnp.int32),
            pltpu.VMEM((CB, EMB_DIM), jnp.float32),
            pltpu.VMEM((CB, EMB_DIM), jnp.float32),
            pltpu.VMEM((BPW,), jnp.float32),
            pltpu.SemaphoreType.DMA,
        ],
    )
    return f(g_idx, c_idx, graph_emb, ctx_emb)


def kernel(g_idx, c_idx, graph_emb, ctx_emb):
    return _pvdbow_scores(g_idx.astype(jnp.int32), c_idx.astype(jnp.int32),
                          graph_emb, ctx_emb)
